# Initial kernel scaffold; baseline (speedup 1.0000x reference)
import jax
import jax.numpy as jnp
from jax.experimental import pallas as pl


def kernel(query, keys, W_q, W_k, segment_offset):
    raise NotImplementedError("write your pallas kernel here")


# speedup vs baseline: 2.9113x; 2.9113x over previous
import functools

import jax
import jax.numpy as jnp
from jax import lax
from jax.experimental import pallas as pl
from jax.experimental.pallas import tpu as pltpu
from jax.experimental.pallas import tpu_sc as plsc

B = 1024
N = 100000
H = 64
TOPK = 16
SCALE_ = 1.0 / (64.0 ** 0.5)
NEG = float("-inf")

BM = 256
BN = 512
GRID_M = B // BM
GRID_N = (N + BN - 1) // BN
GPB = BN // 128
G = GRID_N * GPB
GV = G // 16

LANE = 16


def _sim_kernel(off_ref, q_ref, k_ref, wq_ref, wk_ref, sim_ref, gm_ref):
    n = pl.program_id(1)
    pq = lax.dot_general(q_ref[...], wq_ref[...], (((1,), (1,)), ((), ())),
                         preferred_element_type=jnp.float32)
    pk = lax.dot_general(k_ref[...], wk_ref[...], (((1,), (1,)), ((), ())),
                         preferred_element_type=jnp.float32)
    s = lax.dot_general(pq, pk, (((1,), (1,)), ((), ())),
                        preferred_element_type=jnp.float32) * SCALE_
    off = off_ref[0]
    col = n * BN + lax.broadcasted_iota(jnp.int32, (BM, BN), 1)
    masked = (off > 0) & (off < N) & (col >= off)
    s = jnp.where(masked, NEG, s)
    sim_ref[...] = s
    gm_in = jnp.where(col < N, s, NEG)
    parts = [jnp.max(gm_in[:, g * 128:(g + 1) * 128], axis=1, keepdims=True)
             for g in range(GPB)]
    gm_ref[0, :, :] = jnp.concatenate(parts, axis=1)


def _similarity(query, keys, W_q, W_k, off_arr):
    return pl.pallas_call(
        _sim_kernel,
        grid=(GRID_M, GRID_N),
        in_specs=[
            pl.BlockSpec(memory_space=pltpu.SMEM),
            pl.BlockSpec((BM, H), lambda m, n: (m, 0)),
            pl.BlockSpec((BN, H), lambda m, n: (n, 0)),
            pl.BlockSpec((H, H), lambda m, n: (0, 0)),
            pl.BlockSpec((H, H), lambda m, n: (0, 0)),
        ],
        out_specs=[
            pl.BlockSpec((BM, BN), lambda m, n: (m, n)),
            pl.BlockSpec((1, BM, GPB), lambda m, n: (n, m, 0)),
        ],
        out_shape=[
            jax.ShapeDtypeStruct((B, N), jnp.float32),
            jax.ShapeDtypeStruct((GRID_N, B, GPB), jnp.float32),
        ],
        compiler_params=pltpu.CompilerParams(
            dimension_semantics=("parallel", "parallel")),
    )(off_arr, query, keys, W_q, W_k)


def _merge16(cv, ci, v, vi):
    ka, va = plsc.sort_key_val(v, vi, descending=False)
    m = cv >= ka
    nv = jnp.where(m, cv, ka)
    ni = jnp.where(m, ci, va)
    return plsc.sort_key_val(nv, ni, descending=True)


def _extract(vec, j):
    lanes = lax.iota(jnp.int32, LANE)
    return jnp.sum(jnp.where(lanes == j, vec, 0))


def _topk_body(sim_hbm, gm_hbm, w_out, i_out, gm_v, cand_v, w_buf, i_buf, sem):
    info = plsc.get_sparse_core_info()
    nw = info.num_cores * info.num_subcores
    rpw = B // nw
    wid = lax.axis_index("s") * info.num_cores + lax.axis_index("c")
    base = wid * rpw
    lanes = lax.iota(jnp.int32, LANE)

    def row_body(i, _):
        r = base + i
        r8 = pl.multiple_of(base + (i // 8) * 8, 8)
        rm8 = i % 8
        pltpu.sync_copy(gm_hbm.at[pl.ds(r * G, G)], gm_v)

        def gm_step(j, carry):
            cv, ci, thr = carry
            v = gm_v[pl.ds(j * LANE, LANE)]
            gid = lanes + j * LANE

            def do(cv, ci):
                cv2, ci2 = _merge16(cv, ci, v, gid)
                return cv2, ci2, jnp.min(cv2)

            return lax.cond(jnp.max(v) > thr, do,
                            lambda cv, ci: (cv, ci, thr), cv, ci)

        gvals, gids, _ = lax.fori_loop(
            0, GV, gm_step,
            (jnp.full((LANE,), NEG, jnp.float32),
             jnp.zeros((LANE,), jnp.int32),
             jnp.float32(NEG)))

        copies = []
        for j in range(TOPK):
            gc = jnp.minimum(_extract(gids, j), (N - 1) // 128)
            copies.append(pltpu.async_copy(
                sim_hbm.at[pl.ds(r8, 8), pl.ds(gc * 128, 128)],
                cand_v.at[j], sem))
        for cp in copies:
            cp.wait()

        def grp_step(j, carry):
            cv, ci, thr = carry
            gj = jnp.minimum(_extract(gids, j), (N - 1) // 128)
            gmax = jnp.sum(jnp.where(lanes == j, gvals, 0.0))
            gstart = gj * 128

            def do_group(cv, ci, thr):
                for t in range(8):
                    v = cand_v[j, rm8, pl.ds(t * LANE, LANE)]
                    col = gstart + t * LANE + lanes
                    ok = col < N
                    v = jnp.where(ok, v, NEG)

                    def do(cv, ci):
                        cv2, ci2 = _merge16(cv, ci, v, col)
                        return cv2, ci2, jnp.min(cv2)

                    cv, ci, thr = lax.cond(
                        jnp.max(v) > thr, do,
                        lambda cv, ci: (cv, ci, thr), cv, ci)
                return cv, ci, thr

            return lax.cond(gmax > thr, do_group,
                            lambda cv, ci, thr: (cv, ci, thr), cv, ci, thr)

        tv, ti, _ = lax.fori_loop(
            0, TOPK, grp_step,
            (jnp.full((LANE,), NEG, jnp.float32),
             jnp.zeros((LANE,), jnp.int32),
             jnp.float32(NEG)))

        e = jnp.exp(tv - jnp.max(tv))
        w = e / jnp.sum(e)
        w_buf[pl.ds(i * LANE, LANE)] = w
        i_buf[pl.ds(i * LANE, LANE)] = ti
        return 0

    lax.fori_loop(0, rpw, row_body, 0)
    pltpu.sync_copy(w_buf, w_out.at[pl.ds(base * TOPK, rpw * TOPK)])
    pltpu.sync_copy(i_buf, i_out.at[pl.ds(base * TOPK, rpw * TOPK)])


def _topk_sc(sim, gmax):
    info = plsc.get_sparse_core_info()
    nw = info.num_cores * info.num_subcores
    rpw = B // nw
    mesh = plsc.VectorSubcoreMesh(core_axis_name="c", subcore_axis_name="s")
    fn = pl.kernel(
        _topk_body,
        mesh=mesh,
        out_type=[
            jax.ShapeDtypeStruct((B * TOPK,), jnp.float32),
            jax.ShapeDtypeStruct((B * TOPK,), jnp.int32),
        ],
        scratch_types=[
            pltpu.VMEM((G,), jnp.float32),
            pltpu.VMEM((TOPK, 8, 128), jnp.float32),
            pltpu.VMEM((rpw * TOPK,), jnp.float32),
            pltpu.VMEM((rpw * TOPK,), jnp.int32),
            pltpu.SemaphoreType.DMA,
        ],
        compiler_params=pltpu.CompilerParams(needs_layout_passes=False),
    )
    w, idx = fn(sim, gmax)
    return w.reshape(B, TOPK), idx.reshape(B, TOPK)


def kernel(query, keys, W_q, W_k, segment_offset):
    off_arr = jnp.asarray(segment_offset, jnp.int32).reshape(1)
    sim, gmax3 = _similarity(query, keys, W_q, W_k, off_arr)
    gmax = gmax3.transpose(1, 0, 2).reshape(B * G)
    w, idx = _topk_sc(sim, gmax)
    return (w, idx, sim)


# speedup vs baseline: 4.7872x; 1.6443x over previous
import functools

import jax
import jax.numpy as jnp
from jax import lax
from jax.experimental import pallas as pl
from jax.experimental.pallas import tpu as pltpu
from jax.experimental.pallas import tpu_sc as plsc

B = 1024
N = 100000
H = 64
TOPK = 16
SCALE_ = 1.0 / (64.0 ** 0.5)
NEG = float("-inf")

BM = 512
BN = 1024
GRID_M = B // BM
GRID_N = (N + BN - 1) // BN
GPB = BN // 128
G = GRID_N * GPB
GV = G // 16

LANE = 16


def _sim_kernel(off_ref, q_ref, k_ref, wq_ref, wk_ref, sim_ref, gm_ref):
    n = pl.program_id(1)
    pq = lax.dot_general(q_ref[...], wq_ref[...], (((1,), (1,)), ((), ())),
                         preferred_element_type=jnp.float32)
    pk = lax.dot_general(k_ref[...], wk_ref[...], (((1,), (1,)), ((), ())),
                         preferred_element_type=jnp.float32)
    s = lax.dot_general(pq, pk, (((1,), (1,)), ((), ())),
                        preferred_element_type=jnp.float32) * SCALE_
    off = off_ref[0]
    m_off = jnp.minimum(jnp.where((off > 0) & (off < N), off, N), N)
    touched = (n + 1) * BN > m_off

    def emit(x):
        sim_ref[...] = x
        parts = [jnp.max(x[:, g * 128:(g + 1) * 128], axis=1, keepdims=True)
                 for g in range(GPB)]
        gm_ref[0] = jnp.concatenate(parts, axis=1)

    @pl.when(jnp.logical_not(touched))
    def _():
        emit(s)

    @pl.when(touched)
    def _():
        col = n * BN + lax.broadcasted_iota(jnp.int32, (BM, BN), 1)
        emit(jnp.where(col >= m_off, NEG, s))


def _similarity(query, keys, W_q, W_k, off_arr):
    return pl.pallas_call(
        _sim_kernel,
        grid=(GRID_M, GRID_N),
        in_specs=[
            pl.BlockSpec(memory_space=pltpu.SMEM),
            pl.BlockSpec((BM, H), lambda m, n: (m, 0)),
            pl.BlockSpec((BN, H), lambda m, n: (n, 0)),
            pl.BlockSpec((H, H), lambda m, n: (0, 0)),
            pl.BlockSpec((H, H), lambda m, n: (0, 0)),
        ],
        out_specs=[
            pl.BlockSpec((BM, BN), lambda m, n: (m, n)),
            pl.BlockSpec((1, BM, GPB), lambda m, n: (n, m, 0)),
        ],
        out_shape=[
            jax.ShapeDtypeStruct((B, N), jnp.float32),
            jax.ShapeDtypeStruct((GRID_N, B, GPB), jnp.float32),
        ],
        compiler_params=pltpu.CompilerParams(
            dimension_semantics=("parallel", "parallel")),
    )(off_arr, query, keys, W_q, W_k)


def _merge16(cv, ci, v, vi):
    ka, va = plsc.sort_key_val(v, vi, descending=False)
    m = cv >= ka
    nv = jnp.where(m, cv, ka)
    ni = jnp.where(m, ci, va)
    return plsc.sort_key_val(nv, ni, descending=True)


def _extract(vec, j):
    lanes = lax.iota(jnp.int32, LANE)
    return jnp.sum(jnp.where(lanes == j, vec, 0))


def _topk_body(sim_hbm, gm_hbm, w_out, i_out, gm_v, cand_v, w_buf, i_buf, sem):
    info = plsc.get_sparse_core_info()
    nw = info.num_cores * info.num_subcores
    rpw = B // nw
    wid = lax.axis_index("s") * info.num_cores + lax.axis_index("c")
    base = wid * rpw
    lanes = lax.iota(jnp.int32, LANE)

    def row_body(i, _):
        r = base + i
        r8 = pl.multiple_of(base + (i // 8) * 8, 8)
        rm8 = i % 8
        pltpu.sync_copy(gm_hbm.at[pl.ds(r * G, G)], gm_v)

        def gm_step(j, carry):
            cv, ci, thr = carry
            v = gm_v[pl.ds(j * LANE, LANE)]
            gid = lanes + j * LANE

            def do(cv, ci):
                cv2, ci2 = _merge16(cv, ci, v, gid)
                return cv2, ci2, jnp.min(cv2)

            return lax.cond(jnp.max(v) > thr, do,
                            lambda cv, ci: (cv, ci, thr), cv, ci)

        gvals, gids, _ = lax.fori_loop(
            0, GV, gm_step,
            (jnp.full((LANE,), NEG, jnp.float32),
             jnp.zeros((LANE,), jnp.int32),
             jnp.float32(NEG)))

        copies = []
        for j in range(TOPK):
            gc = jnp.minimum(_extract(gids, j), (N - 1) // 128)
            copies.append(pltpu.async_copy(
                sim_hbm.at[pl.ds(r8, 8), pl.ds(gc * 128, 128)],
                cand_v.at[j], sem))
        for cp in copies:
            cp.wait()

        def grp_step(j, carry):
            cv, ci, thr = carry
            gj = jnp.minimum(_extract(gids, j), (N - 1) // 128)
            gmax = jnp.sum(jnp.where(lanes == j, gvals, 0.0))
            gstart = gj * 128

            def do_group(cv, ci, thr):
                for t in range(8):
                    v = cand_v[j, rm8, pl.ds(t * LANE, LANE)]
                    col = gstart + t * LANE + lanes
                    ok = col < N
                    v = jnp.where(ok, v, NEG)

                    def do(cv, ci):
                        cv2, ci2 = _merge16(cv, ci, v, col)
                        return cv2, ci2, jnp.min(cv2)

                    cv, ci, thr = lax.cond(
                        jnp.max(v) > thr, do,
                        lambda cv, ci: (cv, ci, thr), cv, ci)
                return cv, ci, thr

            return lax.cond(gmax > thr, do_group,
                            lambda cv, ci, thr: (cv, ci, thr), cv, ci, thr)

        tv, ti, _ = lax.fori_loop(
            0, TOPK, grp_step,
            (jnp.full((LANE,), NEG, jnp.float32),
             jnp.zeros((LANE,), jnp.int32),
             jnp.float32(NEG)))

        e = jnp.exp(tv - jnp.max(tv))
        w = e / jnp.sum(e)
        w_buf[pl.ds(i * LANE, LANE)] = w
        i_buf[pl.ds(i * LANE, LANE)] = ti
        return 0

    lax.fori_loop(0, rpw, row_body, 0)
    pltpu.sync_copy(w_buf, w_out.at[pl.ds(base * TOPK, rpw * TOPK)])
    pltpu.sync_copy(i_buf, i_out.at[pl.ds(base * TOPK, rpw * TOPK)])


def _topk_sc(sim, gmax):
    info = plsc.get_sparse_core_info()
    nw = info.num_cores * info.num_subcores
    rpw = B // nw
    mesh = plsc.VectorSubcoreMesh(core_axis_name="c", subcore_axis_name="s")
    fn = pl.kernel(
        _topk_body,
        mesh=mesh,
        out_type=[
            jax.ShapeDtypeStruct((B * TOPK,), jnp.float32),
            jax.ShapeDtypeStruct((B * TOPK,), jnp.int32),
        ],
        scratch_types=[
            pltpu.VMEM((G,), jnp.float32),
            pltpu.VMEM((TOPK, 8, 128), jnp.float32),
            pltpu.VMEM((rpw * TOPK,), jnp.float32),
            pltpu.VMEM((rpw * TOPK,), jnp.int32),
            pltpu.SemaphoreType.DMA,
        ],
        compiler_params=pltpu.CompilerParams(needs_layout_passes=False),
    )
    w, idx = fn(sim, gmax)
    return w.reshape(B, TOPK), idx.reshape(B, TOPK)


def kernel(query, keys, W_q, W_k, segment_offset):
    off_arr = jnp.asarray(segment_offset, jnp.int32).reshape(1)
    sim, gmax3 = _similarity(query, keys, W_q, W_k, off_arr)
    gmax = gmax3.transpose(1, 0, 2).reshape(B * G)
    w, idx = _topk_sc(sim, gmax)
    return (w, idx, sim)


# speedup vs baseline: 5.2768x; 1.1023x over previous
import functools

import jax
import jax.numpy as jnp
from jax import lax
from jax.experimental import pallas as pl
from jax.experimental.pallas import tpu as pltpu
from jax.experimental.pallas import tpu_sc as plsc

B = 1024
N = 100000
H = 64
TOPK = 16
SCALE_ = 1.0 / (64.0 ** 0.5)
NEG = float("-inf")

BM = 512
BN = 1024
GRID_M = B // BM
GRID_N = (N + BN - 1) // BN
GPB = BN // 128
G = GRID_N * GPB
GV = G // 16

LANE = 16


def _sim_kernel(off_ref, q_ref, k_ref, wq_ref, wk_ref, sim_ref, gm_ref, acc_ref):
    n = pl.program_id(1)
    pq = lax.dot_general(q_ref[...], wq_ref[...], (((1,), (1,)), ((), ())),
                         preferred_element_type=jnp.float32)
    pk = lax.dot_general(k_ref[...], wk_ref[...], (((1,), (1,)), ((), ())),
                         preferred_element_type=jnp.float32)
    s = lax.dot_general(pq, pk, (((1,), (1,)), ((), ())),
                        preferred_element_type=jnp.float32) * SCALE_
    off = off_ref[0]
    m_off = jnp.minimum(jnp.where((off > 0) & (off < N), off, N), N)
    touched = (n + 1) * BN > m_off

    def emit(x):
        sim_ref[...] = x
        parts = [jnp.max(x[:, g * 128:(g + 1) * 128], axis=1, keepdims=True)
                 for g in range(GPB)]
        vals = jnp.concatenate(parts, axis=1)
        acc_ref[pl.ds(n * GPB, GPB), :] = vals.T

    @pl.when(jnp.logical_not(touched))
    def _():
        emit(s)

    @pl.when(touched)
    def _():
        col = n * BN + lax.broadcasted_iota(jnp.int32, (BM, BN), 1)
        emit(jnp.where(col >= m_off, NEG, s))

    @pl.when(n == GRID_N - 1)
    def _():
        gm_ref[...] = acc_ref[...].T


def _similarity(query, keys, W_q, W_k, off_arr):
    return pl.pallas_call(
        _sim_kernel,
        grid=(GRID_M, GRID_N),
        in_specs=[
            pl.BlockSpec(memory_space=pltpu.SMEM),
            pl.BlockSpec((BM, H), lambda m, n: (m, 0)),
            pl.BlockSpec((BN, H), lambda m, n: (n, 0)),
            pl.BlockSpec((H, H), lambda m, n: (0, 0)),
            pl.BlockSpec((H, H), lambda m, n: (0, 0)),
        ],
        out_specs=[
            pl.BlockSpec((BM, BN), lambda m, n: (m, n)),
            pl.BlockSpec((BM, G), lambda m, n: (m, 0)),
        ],
        out_shape=[
            jax.ShapeDtypeStruct((B, N), jnp.float32),
            jax.ShapeDtypeStruct((B, G), jnp.float32),
        ],
        scratch_shapes=[pltpu.VMEM((G, BM), jnp.float32)],
        compiler_params=pltpu.CompilerParams(
            dimension_semantics=("parallel", "arbitrary")),
    )(off_arr, query, keys, W_q, W_k)


def _merge16(cv, ci, v, vi):
    ka, va = plsc.sort_key_val(v, vi, descending=False)
    m = cv >= ka
    nv = jnp.where(m, cv, ka)
    ni = jnp.where(m, ci, va)
    return plsc.sort_key_val(nv, ni, descending=True)


def _extract(vec, j):
    lanes = lax.iota(jnp.int32, LANE)
    return jnp.sum(jnp.where(lanes == j, vec, 0))


def _topk_body(sim_hbm, gm_hbm, w_out, i_out, gm_v, cand_v, w_buf, i_buf, sem):
    info = plsc.get_sparse_core_info()
    nw = info.num_cores * info.num_subcores
    rpw = B // nw
    wid = lax.axis_index("s") * info.num_cores + lax.axis_index("c")
    base = wid * rpw
    lanes = lax.iota(jnp.int32, LANE)

    def tile_body(tb, _0):
        r8 = pl.multiple_of(base + tb * 8, 8)
        pltpu.sync_copy(gm_hbm.at[pl.ds(r8, 8), :], gm_v)
        lax.fori_loop(0, 8, functools.partial(row_body, tb), 0)
        return 0

    def row_body(tb, rm8, _0):
        i = tb * 8 + rm8
        r8 = pl.multiple_of(base + tb * 8, 8)

        def gm_step(j, carry):
            cv, ci, thr = carry
            v = gm_v[rm8, pl.ds(j * LANE, LANE)]
            gid = lanes + j * LANE

            def do(cv, ci):
                cv2, ci2 = _merge16(cv, ci, v, gid)
                return cv2, ci2, jnp.min(cv2)

            return lax.cond(jnp.max(v) > thr, do,
                            lambda cv, ci: (cv, ci, thr), cv, ci)

        gvals, gids, _ = lax.fori_loop(
            0, GV, gm_step,
            (jnp.full((LANE,), NEG, jnp.float32),
             jnp.zeros((LANE,), jnp.int32),
             jnp.float32(NEG)))

        copies = []
        for j in range(TOPK):
            gc = jnp.minimum(_extract(gids, j), (N - 1) // 128)
            copies.append(pltpu.async_copy(
                sim_hbm.at[pl.ds(r8, 8), pl.ds(gc * 128, 128)],
                cand_v.at[j], sem))
        for cp in copies:
            cp.wait()

        def grp_step(j, carry):
            cv, ci, thr = carry
            gj = jnp.minimum(_extract(gids, j), (N - 1) // 128)
            gmax = jnp.sum(jnp.where(lanes == j, gvals, 0.0))
            gstart = gj * 128

            def do_group(cv, ci, thr):
                for t in range(8):
                    v = cand_v[j, rm8, pl.ds(t * LANE, LANE)]
                    col = gstart + t * LANE + lanes
                    ok = col < N
                    v = jnp.where(ok, v, NEG)

                    def do(cv, ci):
                        cv2, ci2 = _merge16(cv, ci, v, col)
                        return cv2, ci2, jnp.min(cv2)

                    cv, ci, thr = lax.cond(
                        jnp.max(v) > thr, do,
                        lambda cv, ci: (cv, ci, thr), cv, ci)
                return cv, ci, thr

            return lax.cond(gmax > thr, do_group,
                            lambda cv, ci, thr: (cv, ci, thr), cv, ci, thr)

        tv, ti, _ = lax.fori_loop(
            0, TOPK, grp_step,
            (jnp.full((LANE,), NEG, jnp.float32),
             jnp.zeros((LANE,), jnp.int32),
             jnp.float32(NEG)))

        e = jnp.exp(tv - jnp.max(tv))
        w = e / jnp.sum(e)
        w_buf[pl.ds(i * LANE, LANE)] = w
        i_buf[pl.ds(i * LANE, LANE)] = ti
        return 0

    lax.fori_loop(0, rpw // 8, tile_body, 0)
    pltpu.sync_copy(w_buf, w_out.at[pl.ds(base * TOPK, rpw * TOPK)])
    pltpu.sync_copy(i_buf, i_out.at[pl.ds(base * TOPK, rpw * TOPK)])


def _topk_sc(sim, gmax):
    info = plsc.get_sparse_core_info()
    nw = info.num_cores * info.num_subcores
    rpw = B // nw
    mesh = plsc.VectorSubcoreMesh(core_axis_name="c", subcore_axis_name="s")
    fn = pl.kernel(
        _topk_body,
        mesh=mesh,
        out_type=[
            jax.ShapeDtypeStruct((B * TOPK,), jnp.float32),
            jax.ShapeDtypeStruct((B * TOPK,), jnp.int32),
        ],
        scratch_types=[
            pltpu.VMEM((8, G), jnp.float32),
            pltpu.VMEM((TOPK, 8, 128), jnp.float32),
            pltpu.VMEM((rpw * TOPK,), jnp.float32),
            pltpu.VMEM((rpw * TOPK,), jnp.int32),
            pltpu.SemaphoreType.DMA,
        ],
        compiler_params=pltpu.CompilerParams(needs_layout_passes=False),
    )
    w, idx = fn(sim, gmax)
    return w.reshape(B, TOPK), idx.reshape(B, TOPK)


def kernel(query, keys, W_q, W_k, segment_offset):
    off_arr = jnp.asarray(segment_offset, jnp.int32).reshape(1)
    sim, gmax = _similarity(query, keys, W_q, W_k, off_arr)
    w, idx = _topk_sc(sim, gmax)
    return (w, idx, sim)
